# Initial kernel scaffold; baseline (speedup 1.0000x reference)
import jax
import jax.numpy as jnp
from jax.experimental import pallas as pl


def kernel(x, adj, W_share, b_share, gn_weight, gn_bias, gn_mean_scale, W_mean, b_mean, W_disp, b_disp, W_pi, b_pi):
    raise NotImplementedError("write your pallas kernel here")


# speedup vs baseline: 39.5567x; 39.5567x over previous
import functools

import jax
import jax.numpy as jnp
from jax import lax
from jax.experimental import pallas as pl
from jax.experimental.pallas import tpu as pltpu
from jax.experimental.pallas import tpu_sc as plsc

N = 10000
E = 320000
D = 128
H = 40
HP = 48
NPAD = 10240
NC = 2
NS = 16
NW = NC * NS
CHUNK = 128
CW = (E + NW * CHUNK - 1) // (NW * CHUNK)
CW = CW + (CW % 2)
EPAD = NW * CW * CHUNK
RPT = NPAD // NS

_mesh = plsc.VectorSubcoreMesh(core_axis_name="c", subcore_axis_name="s")
_sc_params = pltpu.CompilerParams(use_tc_tiling_on_sc=False)


@functools.partial(
    pl.kernel,
    out_type=jax.ShapeDtypeStruct((NC, NPAD, 8), jnp.float32),
    mesh=_mesh,
    compiler_params=_sc_params,
    scratch_types=[
        pltpu.VMEM((CW, CHUNK), jnp.int32),
        pltpu.VMEM((CHUNK, 8), jnp.float32),
        pltpu.VMEM_SHARED((NPAD, 8), jnp.float32),
    ],
)
def _deg_kernel(dst_hbm, zeros_hbm, ones_hbm, out_hbm, didx, ones_v, acc):
    cid = lax.axis_index("c")
    sid = lax.axis_index("s")
    wid = sid * NC + cid
    pltpu.sync_copy(dst_hbm.at[pl.ds(wid * CW, CW)], didx)
    pltpu.sync_copy(ones_hbm, ones_v)
    pltpu.sync_copy(zeros_hbm.at[pl.ds(sid * RPT, RPT)],
                    acc.at[pl.ds(sid * RPT, RPT)])
    plsc.subcore_barrier()

    def body(j, carry):
        pltpu.sync_copy(ones_v, acc.at[didx.at[j]], add=True)
        return carry

    lax.fori_loop(0, CW, body, 0)
    plsc.subcore_barrier()
    pltpu.sync_copy(acc.at[pl.ds(sid * RPT, RPT)],
                    out_hbm.at[cid, pl.ds(sid * RPT, RPT)])


@functools.partial(
    pl.kernel,
    out_type=jax.ShapeDtypeStruct((NC, NPAD, HP), jnp.float32),
    mesh=_mesh,
    compiler_params=_sc_params,
    scratch_types=[
        pltpu.VMEM((CW, CHUNK), jnp.int32),
        pltpu.VMEM((CW, CHUNK), jnp.int32),
        pltpu.VMEM((CHUNK, HP), jnp.float32),
        pltpu.VMEM_SHARED((NPAD, HP), jnp.float32),
        pltpu.SemaphoreType.DMA,
    ],
)
def _agg_kernel(src_hbm, dst_hbm, feat_hbm, zeros_hbm, out_hbm,
                sidx, didx, rows, acc, sem):
    cid = lax.axis_index("c")
    sid = lax.axis_index("s")
    wid = sid * NC + cid
    pltpu.sync_copy(src_hbm.at[pl.ds(wid * CW, CW)], sidx)
    pltpu.sync_copy(dst_hbm.at[pl.ds(wid * CW, CW)], didx)
    pltpu.sync_copy(zeros_hbm.at[pl.ds(sid * RPT, RPT)],
                    acc.at[pl.ds(sid * RPT, RPT)])
    plsc.subcore_barrier()

    def body(j, carry):
        pltpu.async_copy(feat_hbm.at[sidx.at[j]], rows, sem).wait()
        pltpu.sync_copy(rows, acc.at[didx.at[j]], add=True)
        return carry

    lax.fori_loop(0, CW, body, 0)
    plsc.subcore_barrier()
    pltpu.sync_copy(acc.at[pl.ds(sid * RPT, RPT)],
                    out_hbm.at[cid, pl.ds(sid * RPT, RPT)])


def _lin_body(x_ref, w_ref, degp_ref, h_ref, dinv_ref):
    h = jnp.dot(x_ref[...], w_ref[...], preferred_element_type=jnp.float32)
    deg = degp_ref[0, :, 0:1] + degp_ref[1, :, 0:1] + 1.0
    dinv = lax.rsqrt(deg)
    h_ref[...] = h * dinv
    dinv_ref[...] = dinv


def _gn_body(aggp_ref, h1p_ref, dinv_ref, bsh_ref, gnw_ref, gnb_ref, gms_ref,
             hp_ref):
    m = (lax.broadcasted_iota(jnp.int32, (NPAD, 1), 0) < N).astype(jnp.float32)
    dinv = dinv_ref[...]
    z = (dinv * (aggp_ref[0] + aggp_ref[1] + h1p_ref[...]) + bsh_ref[...]) * m
    mean = jnp.sum(z, axis=0, keepdims=True) * (1.0 / N)
    c = (z - mean * gms_ref[...]) * m
    var = jnp.sum(c * c, axis=0, keepdims=True) * (1.0 / N)
    hidden = gnw_ref[...] * (c * lax.rsqrt(var + 1e-5)) + gnb_ref[...]
    hp_ref[...] = jnp.maximum(hidden, 0.0) * dinv * m


def _heads_body(aggp_ref, hp_ref, dinv_ref, wm_ref, bm_ref, wd_ref, bd_ref,
                wp_ref, bp_ref, mo_ref, do_ref, po_ref):
    z = dinv_ref[...] * (aggp_ref[0] + aggp_ref[1] + hp_ref[...])
    ym = jnp.dot(z, wm_ref[...], preferred_element_type=jnp.float32) + bm_ref[...]
    mo_ref[...] = jnp.clip(jnp.exp(ym), 1e-5, 1e6)
    yd = jnp.dot(z, wd_ref[...], preferred_element_type=jnp.float32) + bd_ref[...]
    sp = jnp.maximum(yd, 0.0) + jnp.log1p(jnp.exp(-jnp.abs(yd)))
    do_ref[...] = jnp.clip(sp, 1e-4, 1e4)
    yp = jnp.dot(z, wp_ref[...], preferred_element_type=jnp.float32) + bp_ref[...]
    po_ref[...] = 1.0 / (1.0 + jnp.exp(-yp))


def kernel(x, adj, W_share, b_share, gn_weight, gn_bias, gn_mean_scale,
           W_mean, b_mean, W_disp, b_disp, W_pi, b_pi):
    f32 = jnp.float32
    xp = jnp.zeros((NPAD, D), f32).at[:N].set(x)
    wsp = jnp.pad(W_share, ((0, 0), (0, HP - H)))
    bshp = jnp.pad(b_share, (0, HP - H)).reshape(1, HP)
    gnwp = jnp.pad(gn_weight, (0, HP - H)).reshape(1, HP)
    gnbp = jnp.pad(gn_bias, (0, HP - H)).reshape(1, HP)
    gmsp = jnp.pad(gn_mean_scale, (0, HP - H)).reshape(1, HP)
    wmp = jnp.pad(W_mean, ((0, HP - H), (0, 0)))
    wdp = jnp.pad(W_disp, ((0, HP - H), (0, 0)))
    wpp = jnp.pad(W_pi, ((0, HP - H), (0, 0)))
    bm2 = b_mean.reshape(1, D)
    bd2 = b_disp.reshape(1, D)
    bp2 = b_pi.reshape(1, D)
    npadding = EPAD - E
    ii = jnp.arange(npadding, dtype=jnp.int32)
    src2d = jnp.concatenate([adj[0], N + (ii % 16)]).reshape(EPAD // CHUNK, CHUNK)
    dst2d = jnp.concatenate([adj[1], N + (ii % 16)]).reshape(EPAD // CHUNK, CHUNK)
    zeros48 = jnp.zeros((NPAD, HP), f32)
    zeros8 = jnp.zeros((NPAD, 8), f32)
    ones8 = jnp.ones((CHUNK, 8), f32)

    degp = _deg_kernel(dst2d, zeros8, ones8)

    RB = 1024
    h1p, dinv = pl.pallas_call(
        _lin_body,
        grid=(NPAD // RB,),
        in_specs=[
            pl.BlockSpec((RB, D), lambda i: (i, 0)),
            pl.BlockSpec((D, HP), lambda i: (0, 0)),
            pl.BlockSpec((NC, RB, 8), lambda i: (0, i, 0)),
        ],
        out_specs=[
            pl.BlockSpec((RB, HP), lambda i: (i, 0)),
            pl.BlockSpec((RB, 1), lambda i: (i, 0)),
        ],
        out_shape=[
            jax.ShapeDtypeStruct((NPAD, HP), f32),
            jax.ShapeDtypeStruct((NPAD, 1), f32),
        ],
    )(xp, wsp, degp)

    agg1p = _agg_kernel(src2d, dst2d, h1p, zeros48)

    hp = pl.pallas_call(
        _gn_body,
        out_shape=jax.ShapeDtypeStruct((NPAD, HP), f32),
    )(agg1p, h1p, dinv, bshp, gnwp, gnbp, gmsp)

    agg2p = _agg_kernel(src2d, dst2d, hp, zeros48)

    RB6 = 1000
    mean_o, disp_o, pi_o = pl.pallas_call(
        _heads_body,
        grid=(N // RB6,),
        in_specs=[
            pl.BlockSpec((NC, RB6, HP), lambda i: (0, i, 0)),
            pl.BlockSpec((RB6, HP), lambda i: (i, 0)),
            pl.BlockSpec((RB6, 1), lambda i: (i, 0)),
            pl.BlockSpec((HP, D), lambda i: (0, 0)),
            pl.BlockSpec((1, D), lambda i: (0, 0)),
            pl.BlockSpec((HP, D), lambda i: (0, 0)),
            pl.BlockSpec((1, D), lambda i: (0, 0)),
            pl.BlockSpec((HP, D), lambda i: (0, 0)),
            pl.BlockSpec((1, D), lambda i: (0, 0)),
        ],
        out_specs=[
            pl.BlockSpec((RB6, D), lambda i: (i, 0)),
            pl.BlockSpec((RB6, D), lambda i: (i, 0)),
            pl.BlockSpec((RB6, D), lambda i: (i, 0)),
        ],
        out_shape=[
            jax.ShapeDtypeStruct((N, D), f32),
            jax.ShapeDtypeStruct((N, D), f32),
            jax.ShapeDtypeStruct((N, D), f32),
        ],
    )(agg2p, hp, dinv, wmp, bm2, wdp, bd2, wpp, bp2)

    return (mean_o, disp_o, pi_o)


# speedup vs baseline: 52.7911x; 1.3346x over previous
import functools

import jax
import jax.numpy as jnp
from jax import lax
from jax.experimental import pallas as pl
from jax.experimental.pallas import tpu as pltpu
from jax.experimental.pallas import tpu_sc as plsc

N = 10000
E = 320000
D = 128
H = 40
HP = 48
NPAD = 10240
NC = 2
NS = 16
NW = NC * NS
CHUNK = 128
CW = (E + NW * CHUNK - 1) // (NW * CHUNK)
CW = CW + (CW % 2)
EPAD = NW * CW * CHUNK
RPT = NPAD // NS

_mesh = plsc.VectorSubcoreMesh(core_axis_name="c", subcore_axis_name="s")
_sc_params = pltpu.CompilerParams(use_tc_tiling_on_sc=False)


@functools.partial(
    pl.kernel,
    out_type=jax.ShapeDtypeStruct((NC, NPAD, 8), jnp.float32),
    mesh=_mesh,
    compiler_params=_sc_params,
    scratch_types=[
        pltpu.VMEM((CW, CHUNK), jnp.int32),
        pltpu.VMEM((CHUNK, 8), jnp.float32),
        pltpu.VMEM_SHARED((NPAD, 8), jnp.float32),
    ],
)
def _deg_kernel(dst_hbm, zeros_hbm, ones_hbm, out_hbm, didx, ones_v, acc):
    cid = lax.axis_index("c")
    sid = lax.axis_index("s")
    wid = sid * NC + cid
    pltpu.sync_copy(dst_hbm.at[pl.ds(wid * CW, CW)], didx)
    pltpu.sync_copy(ones_hbm, ones_v)
    pltpu.sync_copy(zeros_hbm.at[pl.ds(sid * RPT, RPT)],
                    acc.at[pl.ds(sid * RPT, RPT)])
    plsc.subcore_barrier()

    def body(j, carry):
        pltpu.sync_copy(ones_v, acc.at[didx.at[j]], add=True)
        return carry

    lax.fori_loop(0, CW, body, 0)
    plsc.subcore_barrier()
    pltpu.sync_copy(acc.at[pl.ds(sid * RPT, RPT)],
                    out_hbm.at[cid, pl.ds(sid * RPT, RPT)])


@functools.partial(
    pl.kernel,
    out_type=jax.ShapeDtypeStruct((NC, NPAD, HP), jnp.float32),
    mesh=_mesh,
    compiler_params=_sc_params,
    scratch_types=[
        pltpu.VMEM((CW, CHUNK), jnp.int32),
        pltpu.VMEM((CW, CHUNK), jnp.int32),
        pltpu.VMEM((CHUNK, HP), jnp.float32),
        pltpu.VMEM((CHUNK, HP), jnp.float32),
        pltpu.VMEM_SHARED((NPAD, HP), jnp.float32),
        pltpu.SemaphoreType.DMA,
        pltpu.SemaphoreType.DMA,
    ],
)
def _agg_kernel(src_hbm, dst_hbm, feat_hbm, zeros_hbm, out_hbm,
                sidx, didx, rows_a, rows_b, acc, sem_a, sem_b):
    cid = lax.axis_index("c")
    sid = lax.axis_index("s")
    wid = sid * NC + cid
    pltpu.sync_copy(src_hbm.at[pl.ds(wid * CW, CW)], sidx)
    pltpu.sync_copy(dst_hbm.at[pl.ds(wid * CW, CW)], didx)
    pltpu.sync_copy(zeros_hbm.at[pl.ds(sid * RPT, RPT)],
                    acc.at[pl.ds(sid * RPT, RPT)])
    plsc.subcore_barrier()

    pltpu.async_copy(feat_hbm.at[sidx.at[0]], rows_a, sem_a)

    def body(i, carry):
        j = 2 * i
        pltpu.async_copy(feat_hbm.at[sidx.at[j + 1]], rows_b, sem_b)
        pltpu.make_async_copy(feat_hbm.at[sidx.at[j]], rows_a, sem_a).wait()
        pltpu.sync_copy(rows_a, acc.at[didx.at[j]], add=True)

        @pl.when(j + 2 < CW)
        def _():
            pltpu.async_copy(feat_hbm.at[sidx.at[j + 2]], rows_a, sem_a)

        pltpu.make_async_copy(feat_hbm.at[sidx.at[j + 1]], rows_b, sem_b).wait()
        pltpu.sync_copy(rows_b, acc.at[didx.at[j + 1]], add=True)
        return carry

    lax.fori_loop(0, CW // 2, body, 0)
    plsc.subcore_barrier()
    pltpu.sync_copy(acc.at[pl.ds(sid * RPT, RPT)],
                    out_hbm.at[cid, pl.ds(sid * RPT, RPT)])


def _lin_body(x_ref, w_ref, degp_ref, h_ref, dinv_ref):
    h = jnp.dot(x_ref[...], w_ref[...], preferred_element_type=jnp.float32)
    deg = degp_ref[0, :, 0:1] + degp_ref[1, :, 0:1] + 1.0
    dinv = lax.rsqrt(deg)
    h_ref[...] = h * dinv
    dinv_ref[...] = dinv


def _gn_body(aggp_ref, h1p_ref, dinv_ref, bsh_ref, gnw_ref, gnb_ref, gms_ref,
             hp_ref):
    m = (lax.broadcasted_iota(jnp.int32, (NPAD, 1), 0) < N).astype(jnp.float32)
    dinv = dinv_ref[...]
    z = (dinv * (aggp_ref[0] + aggp_ref[1] + h1p_ref[...]) + bsh_ref[...]) * m
    mean = jnp.sum(z, axis=0, keepdims=True) * (1.0 / N)
    c = (z - mean * gms_ref[...]) * m
    var = jnp.sum(c * c, axis=0, keepdims=True) * (1.0 / N)
    hidden = gnw_ref[...] * (c * lax.rsqrt(var + 1e-5)) + gnb_ref[...]
    hp_ref[...] = jnp.maximum(hidden, 0.0) * dinv * m


def _heads_body(aggp_ref, hp_ref, dinv_ref, wm_ref, bm_ref, wd_ref, bd_ref,
                wp_ref, bp_ref, mo_ref, do_ref, po_ref):
    z = dinv_ref[...] * (aggp_ref[0] + aggp_ref[1] + hp_ref[...])
    ym = jnp.dot(z, wm_ref[...], preferred_element_type=jnp.float32) + bm_ref[...]
    mo_ref[...] = jnp.clip(jnp.exp(ym), 1e-5, 1e6)
    yd = jnp.dot(z, wd_ref[...], preferred_element_type=jnp.float32) + bd_ref[...]
    sp = jnp.maximum(yd, 0.0) + jnp.log1p(jnp.exp(-jnp.abs(yd)))
    do_ref[...] = jnp.clip(sp, 1e-4, 1e4)
    yp = jnp.dot(z, wp_ref[...], preferred_element_type=jnp.float32) + bp_ref[...]
    po_ref[...] = 1.0 / (1.0 + jnp.exp(-yp))


def kernel(x, adj, W_share, b_share, gn_weight, gn_bias, gn_mean_scale,
           W_mean, b_mean, W_disp, b_disp, W_pi, b_pi):
    f32 = jnp.float32
    xp = jnp.zeros((NPAD, D), f32).at[:N].set(x)
    wsp = jnp.pad(W_share, ((0, 0), (0, HP - H)))
    bshp = jnp.pad(b_share, (0, HP - H)).reshape(1, HP)
    gnwp = jnp.pad(gn_weight, (0, HP - H)).reshape(1, HP)
    gnbp = jnp.pad(gn_bias, (0, HP - H)).reshape(1, HP)
    gmsp = jnp.pad(gn_mean_scale, (0, HP - H)).reshape(1, HP)
    wmp = jnp.pad(W_mean, ((0, HP - H), (0, 0)))
    wdp = jnp.pad(W_disp, ((0, HP - H), (0, 0)))
    wpp = jnp.pad(W_pi, ((0, HP - H), (0, 0)))
    bm2 = b_mean.reshape(1, D)
    bd2 = b_disp.reshape(1, D)
    bp2 = b_pi.reshape(1, D)
    npadding = EPAD - E
    ii = jnp.arange(npadding, dtype=jnp.int32)
    src2d = jnp.concatenate([adj[0], N + (ii % 16)]).reshape(EPAD // CHUNK, CHUNK)
    dst2d = jnp.concatenate([adj[1], N + (ii % 16)]).reshape(EPAD // CHUNK, CHUNK)
    zeros48 = jnp.zeros((NPAD, HP), f32)
    zeros8 = jnp.zeros((NPAD, 8), f32)
    ones8 = jnp.ones((CHUNK, 8), f32)

    degp = _deg_kernel(dst2d, zeros8, ones8)

    RB = 1024
    h1p, dinv = pl.pallas_call(
        _lin_body,
        grid=(NPAD // RB,),
        in_specs=[
            pl.BlockSpec((RB, D), lambda i: (i, 0)),
            pl.BlockSpec((D, HP), lambda i: (0, 0)),
            pl.BlockSpec((NC, RB, 8), lambda i: (0, i, 0)),
        ],
        out_specs=[
            pl.BlockSpec((RB, HP), lambda i: (i, 0)),
            pl.BlockSpec((RB, 1), lambda i: (i, 0)),
        ],
        out_shape=[
            jax.ShapeDtypeStruct((NPAD, HP), f32),
            jax.ShapeDtypeStruct((NPAD, 1), f32),
        ],
    )(xp, wsp, degp)

    agg1p = _agg_kernel(src2d, dst2d, h1p, zeros48)

    hp = pl.pallas_call(
        _gn_body,
        out_shape=jax.ShapeDtypeStruct((NPAD, HP), f32),
    )(agg1p, h1p, dinv, bshp, gnwp, gnbp, gmsp)

    agg2p = _agg_kernel(src2d, dst2d, hp, zeros48)

    RB6 = 1000
    mean_o, disp_o, pi_o = pl.pallas_call(
        _heads_body,
        grid=(N // RB6,),
        in_specs=[
            pl.BlockSpec((NC, RB6, HP), lambda i: (0, i, 0)),
            pl.BlockSpec((RB6, HP), lambda i: (i, 0)),
            pl.BlockSpec((RB6, 1), lambda i: (i, 0)),
            pl.BlockSpec((HP, D), lambda i: (0, 0)),
            pl.BlockSpec((1, D), lambda i: (0, 0)),
            pl.BlockSpec((HP, D), lambda i: (0, 0)),
            pl.BlockSpec((1, D), lambda i: (0, 0)),
            pl.BlockSpec((HP, D), lambda i: (0, 0)),
            pl.BlockSpec((1, D), lambda i: (0, 0)),
        ],
        out_specs=[
            pl.BlockSpec((RB6, D), lambda i: (i, 0)),
            pl.BlockSpec((RB6, D), lambda i: (i, 0)),
            pl.BlockSpec((RB6, D), lambda i: (i, 0)),
        ],
        out_shape=[
            jax.ShapeDtypeStruct((N, D), f32),
            jax.ShapeDtypeStruct((N, D), f32),
            jax.ShapeDtypeStruct((N, D), f32),
        ],
    )(agg2p, hp, dinv, wmp, bm2, wdp, bd2, wpp, bp2)

    return (mean_o, disp_o, pi_o)


# speedup vs baseline: 56.2202x; 1.0650x over previous
import functools

import jax
import jax.numpy as jnp
from jax import lax
from jax.experimental import pallas as pl
from jax.experimental.pallas import tpu as pltpu
from jax.experimental.pallas import tpu_sc as plsc

N = 10000
E = 320000
D = 128
H = 40
HP = 48
NPAD = 10240
NC = 2
NS = 16
NW = NC * NS
CHUNK = 128
CW = (E + NW * CHUNK - 1) // (NW * CHUNK)
CW = CW + (CW % 2)
EPAD = NW * CW * CHUNK
RPT = NPAD // NS

_mesh = plsc.VectorSubcoreMesh(core_axis_name="c", subcore_axis_name="s")
_sc_params = pltpu.CompilerParams(use_tc_tiling_on_sc=False)


@functools.partial(
    pl.kernel,
    out_type=jax.ShapeDtypeStruct((NC, NPAD, 8), jnp.float32),
    mesh=_mesh,
    compiler_params=_sc_params,
    scratch_types=[
        pltpu.VMEM((CW, CHUNK), jnp.int32),
        pltpu.VMEM((CHUNK, 8), jnp.float32),
        pltpu.VMEM_SHARED((NPAD, 8), jnp.float32),
    ],
)
def _deg_kernel(dst_hbm, zeros_hbm, ones_hbm, out_hbm, didx, ones_v, acc):
    cid = lax.axis_index("c")
    sid = lax.axis_index("s")
    wid = sid * NC + cid
    pltpu.sync_copy(dst_hbm.at[pl.ds(wid * CW, CW)], didx)
    pltpu.sync_copy(ones_hbm, ones_v)
    pltpu.sync_copy(zeros_hbm.at[pl.ds(sid * RPT, RPT)],
                    acc.at[pl.ds(sid * RPT, RPT)])
    plsc.subcore_barrier()

    def body(j, carry):
        pltpu.sync_copy(ones_v, acc.at[didx.at[j]], add=True)
        return carry

    lax.fori_loop(0, CW, body, 0)
    plsc.subcore_barrier()
    pltpu.sync_copy(acc.at[pl.ds(sid * RPT, RPT)],
                    out_hbm.at[cid, pl.ds(sid * RPT, RPT)])


@functools.partial(
    pl.kernel,
    out_type=jax.ShapeDtypeStruct((NC, NPAD, HP), jnp.float32),
    mesh=_mesh,
    compiler_params=_sc_params,
    scratch_types=[
        pltpu.VMEM((CW, CHUNK), jnp.int32),
        pltpu.VMEM((CW, CHUNK), jnp.int32),
        pltpu.VMEM((CHUNK, HP), jnp.float32),
        pltpu.VMEM((CHUNK, HP), jnp.float32),
        pltpu.VMEM((CHUNK, HP), jnp.float32),
        pltpu.VMEM((CHUNK, HP), jnp.float32),
        pltpu.VMEM_SHARED((NPAD, HP), jnp.float32),
        pltpu.SemaphoreType.DMA,
        pltpu.SemaphoreType.DMA,
        pltpu.SemaphoreType.DMA,
        pltpu.SemaphoreType.DMA,
    ],
)
def _agg_kernel(src_hbm, dst_hbm, feat_hbm, zeros_hbm, out_hbm,
                sidx, didx, rows0, rows1, rows2, rows3, acc,
                sem0, sem1, sem2, sem3):
    cid = lax.axis_index("c")
    sid = lax.axis_index("s")
    wid = sid * NC + cid
    rows = (rows0, rows1, rows2, rows3)
    sems = (sem0, sem1, sem2, sem3)
    pltpu.sync_copy(src_hbm.at[pl.ds(wid * CW, CW)], sidx)
    pltpu.sync_copy(dst_hbm.at[pl.ds(wid * CW, CW)], didx)
    pltpu.sync_copy(zeros_hbm.at[pl.ds(sid * RPT, RPT)],
                    acc.at[pl.ds(sid * RPT, RPT)])
    plsc.subcore_barrier()

    for b in range(4):
        pltpu.async_copy(feat_hbm.at[sidx.at[b]], rows[b], sems[b])

    def body(i, carry):
        for b in range(4):
            j = 4 * i + b
            pltpu.make_async_copy(feat_hbm.at[sidx.at[j]], rows[b],
                                  sems[b]).wait()
            pltpu.sync_copy(rows[b], acc.at[didx.at[j]], add=True)

            @pl.when(j + 4 < CW)
            def _():
                pltpu.async_copy(feat_hbm.at[sidx.at[j + 4]], rows[b], sems[b])

        return carry

    lax.fori_loop(0, CW // 4, body, 0)
    plsc.subcore_barrier()
    pltpu.sync_copy(acc.at[pl.ds(sid * RPT, RPT)],
                    out_hbm.at[cid, pl.ds(sid * RPT, RPT)])


def _lin_body(x_ref, w_ref, degp_ref, h_ref, dinv_ref):
    h = jnp.dot(x_ref[...], w_ref[...], preferred_element_type=jnp.float32)
    deg = degp_ref[0, :, 0:1] + degp_ref[1, :, 0:1] + 1.0
    dinv = lax.rsqrt(deg)
    h_ref[...] = h * dinv
    dinv_ref[...] = dinv


def _gn_body(aggp_ref, h1p_ref, dinv_ref, bsh_ref, gnw_ref, gnb_ref, gms_ref,
             hp_ref):
    m = (lax.broadcasted_iota(jnp.int32, (NPAD, 1), 0) < N).astype(jnp.float32)
    dinv = dinv_ref[...]
    z = (dinv * (aggp_ref[0] + aggp_ref[1] + h1p_ref[...]) + bsh_ref[...]) * m
    mean = jnp.sum(z, axis=0, keepdims=True) * (1.0 / N)
    c = (z - mean * gms_ref[...]) * m
    var = jnp.sum(c * c, axis=0, keepdims=True) * (1.0 / N)
    hidden = gnw_ref[...] * (c * lax.rsqrt(var + 1e-5)) + gnb_ref[...]
    hp_ref[...] = jnp.maximum(hidden, 0.0) * dinv * m


def _heads_body(aggp_ref, hp_ref, dinv_ref, wm_ref, bm_ref, wd_ref, bd_ref,
                wp_ref, bp_ref, mo_ref, do_ref, po_ref):
    z = dinv_ref[...] * (aggp_ref[0] + aggp_ref[1] + hp_ref[...])
    ym = jnp.dot(z, wm_ref[...], preferred_element_type=jnp.float32) + bm_ref[...]
    mo_ref[...] = jnp.clip(jnp.exp(ym), 1e-5, 1e6)
    yd = jnp.dot(z, wd_ref[...], preferred_element_type=jnp.float32) + bd_ref[...]
    sp = jnp.maximum(yd, 0.0) + jnp.log1p(jnp.exp(-jnp.abs(yd)))
    do_ref[...] = jnp.clip(sp, 1e-4, 1e4)
    yp = jnp.dot(z, wp_ref[...], preferred_element_type=jnp.float32) + bp_ref[...]
    po_ref[...] = 1.0 / (1.0 + jnp.exp(-yp))


def kernel(x, adj, W_share, b_share, gn_weight, gn_bias, gn_mean_scale,
           W_mean, b_mean, W_disp, b_disp, W_pi, b_pi):
    f32 = jnp.float32
    xp = jnp.zeros((NPAD, D), f32).at[:N].set(x)
    wsp = jnp.pad(W_share, ((0, 0), (0, HP - H)))
    bshp = jnp.pad(b_share, (0, HP - H)).reshape(1, HP)
    gnwp = jnp.pad(gn_weight, (0, HP - H)).reshape(1, HP)
    gnbp = jnp.pad(gn_bias, (0, HP - H)).reshape(1, HP)
    gmsp = jnp.pad(gn_mean_scale, (0, HP - H)).reshape(1, HP)
    wmp = jnp.pad(W_mean, ((0, HP - H), (0, 0)))
    wdp = jnp.pad(W_disp, ((0, HP - H), (0, 0)))
    wpp = jnp.pad(W_pi, ((0, HP - H), (0, 0)))
    bm2 = b_mean.reshape(1, D)
    bd2 = b_disp.reshape(1, D)
    bp2 = b_pi.reshape(1, D)
    npadding = EPAD - E
    ii = jnp.arange(npadding, dtype=jnp.int32)
    src2d = jnp.concatenate([adj[0], N + (ii % 16)]).reshape(EPAD // CHUNK, CHUNK)
    dst2d = jnp.concatenate([adj[1], N + (ii % 16)]).reshape(EPAD // CHUNK, CHUNK)
    zeros48 = jnp.zeros((NPAD, HP), f32)
    zeros8 = jnp.zeros((NPAD, 8), f32)
    ones8 = jnp.ones((CHUNK, 8), f32)

    degp = _deg_kernel(dst2d, zeros8, ones8)

    RB = 1024
    h1p, dinv = pl.pallas_call(
        _lin_body,
        grid=(NPAD // RB,),
        in_specs=[
            pl.BlockSpec((RB, D), lambda i: (i, 0)),
            pl.BlockSpec((D, HP), lambda i: (0, 0)),
            pl.BlockSpec((NC, RB, 8), lambda i: (0, i, 0)),
        ],
        out_specs=[
            pl.BlockSpec((RB, HP), lambda i: (i, 0)),
            pl.BlockSpec((RB, 1), lambda i: (i, 0)),
        ],
        out_shape=[
            jax.ShapeDtypeStruct((NPAD, HP), f32),
            jax.ShapeDtypeStruct((NPAD, 1), f32),
        ],
    )(xp, wsp, degp)

    agg1p = _agg_kernel(src2d, dst2d, h1p, zeros48)

    hp = pl.pallas_call(
        _gn_body,
        out_shape=jax.ShapeDtypeStruct((NPAD, HP), f32),
    )(agg1p, h1p, dinv, bshp, gnwp, gnbp, gmsp)

    agg2p = _agg_kernel(src2d, dst2d, hp, zeros48)

    RB6 = 1000
    mean_o, disp_o, pi_o = pl.pallas_call(
        _heads_body,
        grid=(N // RB6,),
        in_specs=[
            pl.BlockSpec((NC, RB6, HP), lambda i: (0, i, 0)),
            pl.BlockSpec((RB6, HP), lambda i: (i, 0)),
            pl.BlockSpec((RB6, 1), lambda i: (i, 0)),
            pl.BlockSpec((HP, D), lambda i: (0, 0)),
            pl.BlockSpec((1, D), lambda i: (0, 0)),
            pl.BlockSpec((HP, D), lambda i: (0, 0)),
            pl.BlockSpec((1, D), lambda i: (0, 0)),
            pl.BlockSpec((HP, D), lambda i: (0, 0)),
            pl.BlockSpec((1, D), lambda i: (0, 0)),
        ],
        out_specs=[
            pl.BlockSpec((RB6, D), lambda i: (i, 0)),
            pl.BlockSpec((RB6, D), lambda i: (i, 0)),
            pl.BlockSpec((RB6, D), lambda i: (i, 0)),
        ],
        out_shape=[
            jax.ShapeDtypeStruct((N, D), f32),
            jax.ShapeDtypeStruct((N, D), f32),
            jax.ShapeDtypeStruct((N, D), f32),
        ],
    )(agg2p, hp, dinv, wmp, bm2, wdp, bd2, wpp, bp2)

    return (mean_o, disp_o, pi_o)


# speedup vs baseline: 56.8336x; 1.0109x over previous
import functools

import jax
import jax.numpy as jnp
from jax import lax
from jax.experimental import pallas as pl
from jax.experimental.pallas import tpu as pltpu
from jax.experimental.pallas import tpu_sc as plsc

N = 10000
E = 320000
D = 128
H = 40
HP = 48
NPAD = 10240
NC = 2
NS = 16
NW = NC * NS
CHUNK = 128
CW = (E + NW * CHUNK - 1) // (NW * CHUNK)
CW = CW + (CW % 2)
EPAD = NW * CW * CHUNK
RPT = NPAD // NS

_mesh = plsc.VectorSubcoreMesh(core_axis_name="c", subcore_axis_name="s")
_sc_params = pltpu.CompilerParams(use_tc_tiling_on_sc=False)


@functools.partial(
    pl.kernel,
    out_type=jax.ShapeDtypeStruct((NC, NPAD, 8), jnp.float32),
    mesh=_mesh,
    compiler_params=_sc_params,
    scratch_types=[
        pltpu.VMEM((CW, CHUNK), jnp.int32),
        pltpu.VMEM((CHUNK, 8), jnp.float32),
        pltpu.VMEM_SHARED((NPAD, 8), jnp.float32),
        pltpu.SemaphoreType.DMA,
    ],
)
def _deg_kernel(dst_hbm, zeros_hbm, ones_hbm, out_hbm, didx, ones_v, acc, sem):
    cid = lax.axis_index("c")
    sid = lax.axis_index("s")
    wid = sid * NC + cid
    pltpu.sync_copy(dst_hbm.at[pl.ds(wid * CW, CW)], didx)
    pltpu.sync_copy(ones_hbm, ones_v)
    pltpu.sync_copy(zeros_hbm.at[pl.ds(sid * RPT, RPT)],
                    acc.at[pl.ds(sid * RPT, RPT)])
    plsc.subcore_barrier()

    def batch(bi, carry):
        def fire(k, c):
            pltpu.async_copy(ones_v, acc.at[didx.at[bi * 16 + k]], sem,
                             add=True)
            return c

        lax.fori_loop(0, 16, fire, 0)

        def drain(k, c):
            pltpu.make_async_copy(ones_v, acc.at[didx.at[0]], sem).wait()
            return c

        lax.fori_loop(0, 16, drain, 0)
        return carry

    lax.fori_loop(0, CW // 16, batch, 0)
    plsc.subcore_barrier()
    pltpu.sync_copy(acc.at[pl.ds(sid * RPT, RPT)],
                    out_hbm.at[cid, pl.ds(sid * RPT, RPT)])


@functools.partial(
    pl.kernel,
    out_type=jax.ShapeDtypeStruct((NC, NPAD, HP), jnp.float32),
    mesh=_mesh,
    compiler_params=_sc_params,
    scratch_types=[
        pltpu.VMEM((CW, CHUNK), jnp.int32),
        pltpu.VMEM((CW, CHUNK), jnp.int32),
        [pltpu.VMEM((CHUNK, HP), jnp.float32) for _ in range(8)],
        pltpu.VMEM_SHARED((NPAD, HP), jnp.float32),
        [pltpu.SemaphoreType.DMA for _ in range(8)],
        [pltpu.SemaphoreType.DMA for _ in range(8)],
    ],
)
def _agg_kernel(src_hbm, dst_hbm, feat_hbm, zeros_hbm, out_hbm,
                sidx, didx, rows, acc, gsems, ssems):
    cid = lax.axis_index("c")
    sid = lax.axis_index("s")
    wid = sid * NC + cid
    pltpu.sync_copy(src_hbm.at[pl.ds(wid * CW, CW)], sidx)
    pltpu.sync_copy(dst_hbm.at[pl.ds(wid * CW, CW)], didx)
    @pl.when(cid == 0)
    def _():
        pltpu.sync_copy(feat_hbm.at[pl.ds(sid * RPT, RPT)],
                        acc.at[pl.ds(sid * RPT, RPT)])

    @pl.when(cid != 0)
    def _():
        pltpu.sync_copy(zeros_hbm.at[pl.ds(sid * RPT, RPT)],
                        acc.at[pl.ds(sid * RPT, RPT)])

    plsc.subcore_barrier()

    for b in range(4):
        pltpu.async_copy(feat_hbm.at[sidx.at[b]], rows[b], gsems[b])

    def body(i, carry):
        for bb in range(8):
            j = 8 * i + bb
            b = bb
            pltpu.make_async_copy(feat_hbm.at[sidx.at[j]], rows[b],
                                  gsems[b]).wait()
            pltpu.async_copy(rows[b], acc.at[didx.at[j]], ssems[b], add=True)
            b4 = (bb + 4) % 8

            @pl.when(j >= 4)
            def _():
                pltpu.make_async_copy(rows[b4], acc.at[didx.at[j - 4]],
                                      ssems[b4]).wait()

            @pl.when(j + 4 < CW)
            def _():
                pltpu.async_copy(feat_hbm.at[sidx.at[j + 4]], rows[b4],
                                 gsems[b4])

        return carry

    lax.fori_loop(0, CW // 8, body, 0)
    for bb in range(4, 8):
        pltpu.make_async_copy(rows[bb], acc.at[didx.at[CW - 8 + bb]],
                              ssems[bb]).wait()
    plsc.subcore_barrier()
    pltpu.sync_copy(acc.at[pl.ds(sid * RPT, RPT)],
                    out_hbm.at[cid, pl.ds(sid * RPT, RPT)])


def _lin_body(x_ref, w_ref, degp_ref, h_ref, dinv_ref):
    h = jnp.dot(x_ref[...], w_ref[...], preferred_element_type=jnp.float32)
    deg = degp_ref[0, :, 0:1] + degp_ref[1, :, 0:1] + 1.0
    dinv = lax.rsqrt(deg)
    h_ref[...] = h * dinv
    dinv_ref[...] = dinv


def _gn_body(aggp_ref, dinv_ref, bsh_ref, gnw_ref, gnb_ref, gms_ref,
             hp_ref):
    m = (lax.broadcasted_iota(jnp.int32, (NPAD, 1), 0) < N).astype(jnp.float32)
    dinv = dinv_ref[...]
    z = (dinv * (aggp_ref[0] + aggp_ref[1]) + bsh_ref[...]) * m
    mean = jnp.sum(z, axis=0, keepdims=True) * (1.0 / N)
    c = (z - mean * gms_ref[...]) * m
    var = jnp.sum(c * c, axis=0, keepdims=True) * (1.0 / N)
    hidden = gnw_ref[...] * (c * lax.rsqrt(var + 1e-5)) + gnb_ref[...]
    hp_ref[...] = jnp.maximum(hidden, 0.0) * dinv * m


def _heads_body(aggp_ref, dinv_ref, wm_ref, bm_ref, wd_ref, bd_ref,
                wp_ref, bp_ref, mo_ref, do_ref, po_ref):
    z = dinv_ref[...] * (aggp_ref[0] + aggp_ref[1])
    ym = jnp.dot(z, wm_ref[...], preferred_element_type=jnp.float32) + bm_ref[...]
    mo_ref[...] = jnp.clip(jnp.exp(ym), 1e-5, 1e6)
    yd = jnp.dot(z, wd_ref[...], preferred_element_type=jnp.float32) + bd_ref[...]
    sp = jnp.maximum(yd, 0.0) + jnp.log1p(jnp.exp(-jnp.abs(yd)))
    do_ref[...] = jnp.clip(sp, 1e-4, 1e4)
    yp = jnp.dot(z, wp_ref[...], preferred_element_type=jnp.float32) + bp_ref[...]
    po_ref[...] = 1.0 / (1.0 + jnp.exp(-yp))


def kernel(x, adj, W_share, b_share, gn_weight, gn_bias, gn_mean_scale,
           W_mean, b_mean, W_disp, b_disp, W_pi, b_pi):
    f32 = jnp.float32
    xp = jnp.zeros((NPAD, D), f32).at[:N].set(x)
    wsp = jnp.pad(W_share, ((0, 0), (0, HP - H)))
    bshp = jnp.pad(b_share, (0, HP - H)).reshape(1, HP)
    gnwp = jnp.pad(gn_weight, (0, HP - H)).reshape(1, HP)
    gnbp = jnp.pad(gn_bias, (0, HP - H)).reshape(1, HP)
    gmsp = jnp.pad(gn_mean_scale, (0, HP - H)).reshape(1, HP)
    wmp = jnp.pad(W_mean, ((0, HP - H), (0, 0)))
    wdp = jnp.pad(W_disp, ((0, HP - H), (0, 0)))
    wpp = jnp.pad(W_pi, ((0, HP - H), (0, 0)))
    bm2 = b_mean.reshape(1, D)
    bd2 = b_disp.reshape(1, D)
    bp2 = b_pi.reshape(1, D)
    npadding = EPAD - E
    ii = jnp.arange(npadding, dtype=jnp.int32)
    src2d = jnp.concatenate([adj[0], N + (ii % 16)]).reshape(EPAD // CHUNK, CHUNK)
    dst2d = jnp.concatenate([adj[1], N + (ii % 16)]).reshape(EPAD // CHUNK, CHUNK)
    zeros48 = jnp.zeros((NPAD, HP), f32)
    zeros8 = jnp.zeros((NPAD, 8), f32)
    ones8 = jnp.ones((CHUNK, 8), f32)

    degp = _deg_kernel(dst2d, zeros8, ones8)

    RB = 1024
    h1p, dinv = pl.pallas_call(
        _lin_body,
        grid=(NPAD // RB,),
        in_specs=[
            pl.BlockSpec((RB, D), lambda i: (i, 0)),
            pl.BlockSpec((D, HP), lambda i: (0, 0)),
            pl.BlockSpec((NC, RB, 8), lambda i: (0, i, 0)),
        ],
        out_specs=[
            pl.BlockSpec((RB, HP), lambda i: (i, 0)),
            pl.BlockSpec((RB, 1), lambda i: (i, 0)),
        ],
        out_shape=[
            jax.ShapeDtypeStruct((NPAD, HP), f32),
            jax.ShapeDtypeStruct((NPAD, 1), f32),
        ],
    )(xp, wsp, degp)

    agg1p = _agg_kernel(src2d, dst2d, h1p, zeros48)

    hp = pl.pallas_call(
        _gn_body,
        out_shape=jax.ShapeDtypeStruct((NPAD, HP), f32),
    )(agg1p, dinv, bshp, gnwp, gnbp, gmsp)

    agg2p = _agg_kernel(src2d, dst2d, hp, zeros48)

    RB6 = 1000
    mean_o, disp_o, pi_o = pl.pallas_call(
        _heads_body,
        grid=(N // RB6,),
        in_specs=[
            pl.BlockSpec((NC, RB6, HP), lambda i: (0, i, 0)),
            pl.BlockSpec((RB6, 1), lambda i: (i, 0)),
            pl.BlockSpec((HP, D), lambda i: (0, 0)),
            pl.BlockSpec((1, D), lambda i: (0, 0)),
            pl.BlockSpec((HP, D), lambda i: (0, 0)),
            pl.BlockSpec((1, D), lambda i: (0, 0)),
            pl.BlockSpec((HP, D), lambda i: (0, 0)),
            pl.BlockSpec((1, D), lambda i: (0, 0)),
        ],
        out_specs=[
            pl.BlockSpec((RB6, D), lambda i: (i, 0)),
            pl.BlockSpec((RB6, D), lambda i: (i, 0)),
            pl.BlockSpec((RB6, D), lambda i: (i, 0)),
        ],
        out_shape=[
            jax.ShapeDtypeStruct((N, D), f32),
            jax.ShapeDtypeStruct((N, D), f32),
            jax.ShapeDtypeStruct((N, D), f32),
        ],
    )(agg2p, dinv, wmp, bm2, wdp, bd2, wpp, bp2)

    return (mean_o, disp_o, pi_o)


# speedup vs baseline: 57.9195x; 1.0191x over previous
import functools

import jax
import jax.numpy as jnp
from jax import lax
from jax.experimental import pallas as pl
from jax.experimental.pallas import tpu as pltpu
from jax.experimental.pallas import tpu_sc as plsc

N = 10000
E = 320000
D = 128
H = 40
HP = 48
NPAD = 10240
NC = 2
NS = 16
NW = NC * NS
CHUNK = 128
CW = (E + NW * CHUNK - 1) // (NW * CHUNK)
CW = CW + (CW % 2)
EPAD = NW * CW * CHUNK
RPT = NPAD // NS

_mesh = plsc.VectorSubcoreMesh(core_axis_name="c", subcore_axis_name="s")
_sc_params = pltpu.CompilerParams(use_tc_tiling_on_sc=False)


@functools.partial(
    pl.kernel,
    out_type=jax.ShapeDtypeStruct((NC, NPAD, 8), jnp.float32),
    mesh=_mesh,
    compiler_params=_sc_params,
    scratch_types=[
        pltpu.VMEM((CW, CHUNK), jnp.int32),
        pltpu.VMEM((CHUNK, 8), jnp.float32),
        pltpu.VMEM_SHARED((NPAD, 8), jnp.float32),
        pltpu.SemaphoreType.DMA,
    ],
)
def _deg_kernel(dst_hbm, zeros_hbm, ones_hbm, out_hbm, didx, ones_v, acc, sem):
    cid = lax.axis_index("c")
    sid = lax.axis_index("s")
    wid = sid * NC + cid
    pltpu.sync_copy(dst_hbm.at[pl.ds(wid * CW, CW)], didx)
    pltpu.sync_copy(ones_hbm, ones_v)
    pltpu.sync_copy(zeros_hbm.at[pl.ds(sid * RPT, RPT)],
                    acc.at[pl.ds(sid * RPT, RPT)])
    plsc.subcore_barrier()

    def batch(bi, carry):
        def fire(k, c):
            pltpu.async_copy(ones_v, acc.at[didx.at[bi * 16 + k]], sem,
                             add=True)
            return c

        lax.fori_loop(0, 16, fire, 0)

        def drain(k, c):
            pltpu.make_async_copy(ones_v, acc.at[didx.at[0]], sem).wait()
            return c

        lax.fori_loop(0, 16, drain, 0)
        return carry

    lax.fori_loop(0, CW // 16, batch, 0)
    plsc.subcore_barrier()
    pltpu.sync_copy(acc.at[pl.ds(sid * RPT, RPT)],
                    out_hbm.at[cid, pl.ds(sid * RPT, RPT)])


@functools.partial(
    pl.kernel,
    out_type=jax.ShapeDtypeStruct((NC, NPAD, HP), jnp.float32),
    mesh=_mesh,
    compiler_params=_sc_params,
    scratch_types=[
        pltpu.VMEM((CW, CHUNK), jnp.int32),
        pltpu.VMEM((CW, CHUNK), jnp.int32),
        [pltpu.VMEM((CHUNK, HP), jnp.float32) for _ in range(4)],
        pltpu.VMEM_SHARED((NPAD, HP), jnp.float32),
        pltpu.VMEM_SHARED((NPAD, HP), jnp.float32),
        [pltpu.SemaphoreType.DMA for _ in range(4)],
    ],
)
def _agg_kernel(src_hbm, dst_hbm, feat_hbm, zeros_hbm, out_hbm,
                sidx, didx, rows, feat_sh, acc, gsems):
    cid = lax.axis_index("c")
    sid = lax.axis_index("s")
    wid = sid * NC + cid
    pltpu.sync_copy(src_hbm.at[pl.ds(wid * CW, CW)], sidx)
    pltpu.sync_copy(dst_hbm.at[pl.ds(wid * CW, CW)], didx)
    pltpu.sync_copy(feat_hbm.at[pl.ds(sid * RPT, RPT)],
                    feat_sh.at[pl.ds(sid * RPT, RPT)])

    @pl.when(cid == 0)
    def _():
        pltpu.sync_copy(feat_hbm.at[pl.ds(sid * RPT, RPT)],
                        acc.at[pl.ds(sid * RPT, RPT)])

    @pl.when(cid != 0)
    def _():
        pltpu.sync_copy(zeros_hbm.at[pl.ds(sid * RPT, RPT)],
                        acc.at[pl.ds(sid * RPT, RPT)])

    plsc.subcore_barrier()

    for b in range(4):
        pltpu.async_copy(feat_sh.at[sidx.at[b]], rows[b], gsems[b])

    def body(i, carry):
        for b in range(4):
            j = 4 * i + b
            pltpu.make_async_copy(feat_sh.at[sidx.at[j]], rows[b],
                                  gsems[b]).wait()
            pltpu.sync_copy(rows[b], acc.at[didx.at[j]], add=True)

            @pl.when(j + 4 < CW)
            def _():
                pltpu.async_copy(feat_sh.at[sidx.at[j + 4]], rows[b], gsems[b])

        return carry

    lax.fori_loop(0, CW // 4, body, 0)
    plsc.subcore_barrier()
    pltpu.sync_copy(acc.at[pl.ds(sid * RPT, RPT)],
                    out_hbm.at[cid, pl.ds(sid * RPT, RPT)])


def _lin_body(x_ref, w_ref, degp_ref, h_ref, dinv_ref):
    h = jnp.dot(x_ref[...], w_ref[...], preferred_element_type=jnp.float32)
    deg = degp_ref[0, :, 0:1] + degp_ref[1, :, 0:1] + 1.0
    dinv = lax.rsqrt(deg)
    h_ref[...] = h * dinv
    dinv_ref[...] = dinv


def _gn_body(aggp_ref, dinv_ref, bsh_ref, gnw_ref, gnb_ref, gms_ref,
             hp_ref):
    m = (lax.broadcasted_iota(jnp.int32, (NPAD, 1), 0) < N).astype(jnp.float32)
    dinv = dinv_ref[...]
    z = (dinv * (aggp_ref[0] + aggp_ref[1]) + bsh_ref[...]) * m
    mean = jnp.sum(z, axis=0, keepdims=True) * (1.0 / N)
    c = (z - mean * gms_ref[...]) * m
    var = jnp.sum(c * c, axis=0, keepdims=True) * (1.0 / N)
    hidden = gnw_ref[...] * (c * lax.rsqrt(var + 1e-5)) + gnb_ref[...]
    hp_ref[...] = jnp.maximum(hidden, 0.0) * dinv * m


def _heads_body(aggp_ref, dinv_ref, wm_ref, bm_ref, wd_ref, bd_ref,
                wp_ref, bp_ref, mo_ref, do_ref, po_ref):
    z = dinv_ref[...] * (aggp_ref[0] + aggp_ref[1])
    ym = jnp.dot(z, wm_ref[...], preferred_element_type=jnp.float32) + bm_ref[...]
    mo_ref[...] = jnp.clip(jnp.exp(ym), 1e-5, 1e6)
    yd = jnp.dot(z, wd_ref[...], preferred_element_type=jnp.float32) + bd_ref[...]
    sp = jnp.maximum(yd, 0.0) + jnp.log1p(jnp.exp(-jnp.abs(yd)))
    do_ref[...] = jnp.clip(sp, 1e-4, 1e4)
    yp = jnp.dot(z, wp_ref[...], preferred_element_type=jnp.float32) + bp_ref[...]
    po_ref[...] = 1.0 / (1.0 + jnp.exp(-yp))


def kernel(x, adj, W_share, b_share, gn_weight, gn_bias, gn_mean_scale,
           W_mean, b_mean, W_disp, b_disp, W_pi, b_pi):
    f32 = jnp.float32
    xp = jnp.zeros((NPAD, D), f32).at[:N].set(x)
    wsp = jnp.pad(W_share, ((0, 0), (0, HP - H)))
    bshp = jnp.pad(b_share, (0, HP - H)).reshape(1, HP)
    gnwp = jnp.pad(gn_weight, (0, HP - H)).reshape(1, HP)
    gnbp = jnp.pad(gn_bias, (0, HP - H)).reshape(1, HP)
    gmsp = jnp.pad(gn_mean_scale, (0, HP - H)).reshape(1, HP)
    wmp = jnp.pad(W_mean, ((0, HP - H), (0, 0)))
    wdp = jnp.pad(W_disp, ((0, HP - H), (0, 0)))
    wpp = jnp.pad(W_pi, ((0, HP - H), (0, 0)))
    bm2 = b_mean.reshape(1, D)
    bd2 = b_disp.reshape(1, D)
    bp2 = b_pi.reshape(1, D)
    npadding = EPAD - E
    ii = jnp.arange(npadding, dtype=jnp.int32)
    src2d = jnp.concatenate([adj[0], N + (ii % 16)]).reshape(EPAD // CHUNK, CHUNK)
    dst2d = jnp.concatenate([adj[1], N + (ii % 16)]).reshape(EPAD // CHUNK, CHUNK)
    zeros48 = jnp.zeros((NPAD, HP), f32)
    zeros8 = jnp.zeros((NPAD, 8), f32)
    ones8 = jnp.ones((CHUNK, 8), f32)

    degp = _deg_kernel(dst2d, zeros8, ones8)

    RB = 1024
    h1p, dinv = pl.pallas_call(
        _lin_body,
        grid=(NPAD // RB,),
        in_specs=[
            pl.BlockSpec((RB, D), lambda i: (i, 0)),
            pl.BlockSpec((D, HP), lambda i: (0, 0)),
            pl.BlockSpec((NC, RB, 8), lambda i: (0, i, 0)),
        ],
        out_specs=[
            pl.BlockSpec((RB, HP), lambda i: (i, 0)),
            pl.BlockSpec((RB, 1), lambda i: (i, 0)),
        ],
        out_shape=[
            jax.ShapeDtypeStruct((NPAD, HP), f32),
            jax.ShapeDtypeStruct((NPAD, 1), f32),
        ],
    )(xp, wsp, degp)

    agg1p = _agg_kernel(src2d, dst2d, h1p, zeros48)

    hp = pl.pallas_call(
        _gn_body,
        out_shape=jax.ShapeDtypeStruct((NPAD, HP), f32),
    )(agg1p, dinv, bshp, gnwp, gnbp, gmsp)

    agg2p = _agg_kernel(src2d, dst2d, hp, zeros48)

    RB6 = 1000
    mean_o, disp_o, pi_o = pl.pallas_call(
        _heads_body,
        grid=(N // RB6,),
        in_specs=[
            pl.BlockSpec((NC, RB6, HP), lambda i: (0, i, 0)),
            pl.BlockSpec((RB6, 1), lambda i: (i, 0)),
            pl.BlockSpec((HP, D), lambda i: (0, 0)),
            pl.BlockSpec((1, D), lambda i: (0, 0)),
            pl.BlockSpec((HP, D), lambda i: (0, 0)),
            pl.BlockSpec((1, D), lambda i: (0, 0)),
            pl.BlockSpec((HP, D), lambda i: (0, 0)),
            pl.BlockSpec((1, D), lambda i: (0, 0)),
        ],
        out_specs=[
            pl.BlockSpec((RB6, D), lambda i: (i, 0)),
            pl.BlockSpec((RB6, D), lambda i: (i, 0)),
            pl.BlockSpec((RB6, D), lambda i: (i, 0)),
        ],
        out_shape=[
            jax.ShapeDtypeStruct((N, D), f32),
            jax.ShapeDtypeStruct((N, D), f32),
            jax.ShapeDtypeStruct((N, D), f32),
        ],
    )(agg2p, dinv, wmp, bm2, wdp, bd2, wpp, bp2)

    return (mean_o, disp_o, pi_o)
